# baseline (speedup 1.0000x reference)
import jax
import jax.numpy as jnp
import numpy as np
from jax.experimental import pallas as pl
from jax.experimental.pallas import tpu as pltpu

_BB = 8
_FC_BM = 1024
_FC_KC = 2
_GS = (0, 128, 384, 512)


def _conv_fused_kernel(x_ref, w1_ref, w2_ref, b12_ref, sb1_ref,
                       sb2_ref, o_ref):
    BB = x_ref.shape[0]
    M = BB * 32
    x = x_ref[...].astype(jnp.bfloat16)
    xcat = jnp.concatenate([x[:, 0], x[:, 1], x[:, 2]],
                           axis=2).reshape(M, 96)

    m1 = jnp.dot(sb1_ref[...], xcat,
                 preferred_element_type=jnp.float32).astype(jnp.bfloat16)
    acc = jnp.dot(m1[0:M], w1_ref[0], preferred_element_type=jnp.float32)
    acc += jnp.dot(xcat, w1_ref[1], preferred_element_type=jnp.float32)
    acc += jnp.dot(m1[M:2 * M], w1_ref[2],
                   preferred_element_type=jnp.float32)
    r1 = jnp.maximum(acc + b12_ref[0:1], 0.0).astype(jnp.bfloat16)

    m2 = jnp.dot(sb2_ref[...], r1,
                 preferred_element_type=jnp.float32).astype(jnp.bfloat16)
    for g in range(4):
        s = _GS[g]
        a2 = None
        for kh in range(3):
            d = jnp.dot(m2[kh * M:(kh + 1) * M, s:s + 512], w2_ref[kh, g],
                        preferred_element_type=jnp.float32)
            a2 = d if a2 is None else a2 + d
        r2 = jnp.maximum(a2 + b12_ref[1:2, g * 256:(g + 1) * 256], 0.0)
        hm = jnp.maximum(r2[0:M // 2], r2[M // 2:M])
        sh = jnp.concatenate([hm[:, 32:], hm[:, :32]], axis=1)
        o_ref[:, :, g * 256:(g + 1) * 256] = jnp.maximum(hm, sh).astype(
            o_ref.dtype).reshape(16, BB, 256)


def _fc_fused_kernel(x_ref, w3_ref, b3_ref, w4_ref, b4_ref, o_ref, acc_ref):
    k = pl.program_id(1)

    @pl.when(k == 0)
    def _():
        acc_ref[...] = jnp.zeros_like(acc_ref)

    part = jnp.dot(x_ref[0], w3_ref[0], preferred_element_type=jnp.float32)
    for c in range(1, _FC_KC):
        part += jnp.dot(x_ref[c], w3_ref[c],
                        preferred_element_type=jnp.float32)
    acc_ref[...] += part

    @pl.when(k == pl.num_programs(1) - 1)
    def _():
        h = jnp.maximum(acc_ref[...] + b3_ref[...], 0.0).astype(jnp.bfloat16)
        o_ref[...] = (jnp.dot(h, w4_ref[...],
                              preferred_element_type=jnp.float32)
                      + b4_ref[...]).astype(o_ref.dtype)


def _conv_stage(x_nchw, w1L, w2G, b12, sb1, sb2):
    B = x_nchw.shape[0]
    return pl.pallas_call(
        _conv_fused_kernel,
        out_shape=jax.ShapeDtypeStruct((16, B, 1024), jnp.bfloat16),
        grid_spec=pltpu.PrefetchScalarGridSpec(
            num_scalar_prefetch=0,
            grid=(B // _BB,),
            in_specs=[
                pl.BlockSpec((_BB, 3, 32, 32), lambda i: (i, 0, 0, 0)),
                pl.BlockSpec((3, 96, 1024), lambda i: (0, 0, 0)),
                pl.BlockSpec((3, 4, 512, 256), lambda i: (0, 0, 0, 0)),
                pl.BlockSpec((2, 1024), lambda i: (0, 0)),
                pl.BlockSpec((2 * _BB * 32, _BB * 32), lambda i: (0, 0)),
                pl.BlockSpec((3 * _BB * 32, _BB * 32), lambda i: (0, 0)),
            ],
            out_specs=pl.BlockSpec((16, _BB, 1024), lambda i: (0, i, 0)),
        ),
        compiler_params=pltpu.CompilerParams(
            dimension_semantics=("parallel",)),
    )(x_nchw, w1L, w2G, b12, sb1, sb2)


def _fc_stage(feats, w3, b3, w4, b4):
    KC, B, _ = feats.shape
    N3 = w3.shape[2]
    N4 = w4.shape[1]
    return pl.pallas_call(
        _fc_fused_kernel,
        out_shape=jax.ShapeDtypeStruct((B, N4), jnp.float32),
        grid_spec=pltpu.PrefetchScalarGridSpec(
            num_scalar_prefetch=0,
            grid=(B // _FC_BM, KC // _FC_KC),
            in_specs=[
                pl.BlockSpec((_FC_KC, _FC_BM, 1024), lambda i, k: (k, i, 0)),
                pl.BlockSpec((_FC_KC, 1024, N3), lambda i, k: (k, 0, 0)),
                pl.BlockSpec((1, N3), lambda i, k: (0, 0)),
                pl.BlockSpec((N3, N4), lambda i, k: (0, 0)),
                pl.BlockSpec((1, N4), lambda i, k: (0, 0)),
            ],
            out_specs=pl.BlockSpec((_FC_BM, N4), lambda i, k: (i, 0)),
            scratch_shapes=[pltpu.VMEM((_FC_BM, N3), jnp.float32)],
        ),
        compiler_params=pltpu.CompilerParams(
            dimension_semantics=("parallel", "arbitrary")),
    )(feats, w3, b3, w4, b4)


def _shift_consts():
    m = _BB * 32
    sb1 = np.stack([np.kron(np.eye(_BB, dtype=np.float32),
                            np.eye(32, k=kh - 1, dtype=np.float32))
                    for kh in range(3)])
    j = np.arange(m)
    half = j % (m // 2)
    srcrow = (half % _BB) * 32 + (half // _BB) * 2 + j // (m // 2)
    perm = np.zeros((m, m), np.float32)
    perm[j, srcrow] = 1.0
    sb2 = np.einsum('jr,xrs->xjs', perm, sb1)
    sb1_02 = np.concatenate([sb1[0], sb1[2]], axis=0)
    sb2_all = sb2.reshape(3 * m, m)
    return sb1_02, sb2_all


_SB1, _SB2 = _shift_consts()


def _banded(w_taps, cin, row_order):
    wr = w_taps.reshape(3, 3, cin, 32)
    d = jnp.stack([jnp.eye(32, k=1 - kw, dtype=w_taps.dtype)
                   for kw in range(3)])
    wl = jnp.einsum(f'xab,hxio->h{row_order}bo', d, wr)
    return wl.reshape(3, cin * 32, 1024).astype(jnp.bfloat16)


def kernel(conv1_w, conv1_b, conv2_w, conv2_b, fc3_w, fc3_b, fc4_w, fc4_b,
           x_nchw):
    B = x_nchw.shape[0]
    w1L = _banded(conv1_w, 3, 'ia')
    w2L = _banded(conv2_w, 32, 'ai')
    ridx = np.asarray(_GS)[:, None] + np.arange(512)[None, :]
    cidx = 256 * np.arange(4)[:, None] + np.arange(256)[None, :]
    w2G = w2L[:, ridx[:, :, None], cidx[:, None, :]]
    b12 = jnp.tile(jnp.stack([conv1_b, conv2_b]), (1, 32))
    sb1 = jnp.asarray(_SB1, jnp.bfloat16)
    sb2 = jnp.asarray(_SB2, jnp.bfloat16)

    feats = _conv_stage(x_nchw, w1L, w2G, b12, sb1, sb2)

    wt = fc3_w.reshape(16, 16, 32, 512)
    w3w = jnp.stack([wt, jnp.zeros_like(wt)], axis=2).reshape(16, 1024, 512)

    return _fc_stage(feats, w3w.astype(jnp.bfloat16), fc3_b.reshape(1, 512),
                     fc4_w.astype(jnp.bfloat16), fc4_b.reshape(1, 10))


# speedup vs baseline: 33.8615x; 33.8615x over previous
import jax
import jax.numpy as jnp
import numpy as np
from jax.experimental import pallas as pl
from jax.experimental.pallas import tpu as pltpu

_BB = 8
_FC_BM = 1024
_FC_KC = 2
_GS = (0, 128, 384, 512)


def _conv_fused_kernel(x_ref, w1_ref, w2_ref, b12_ref, sb1_ref,
                       sb2_ref, o_ref):
    BB = x_ref.shape[0]
    M = BB * 32
    x = x_ref[...].astype(jnp.bfloat16)
    xcat = jnp.concatenate([x[:, 0], x[:, 1], x[:, 2]],
                           axis=2).reshape(M, 96)

    m1 = jnp.dot(sb1_ref[...], xcat,
                 preferred_element_type=jnp.float32).astype(jnp.bfloat16)
    acc = jnp.dot(m1[0:M], w1_ref[0], preferred_element_type=jnp.float32)
    acc += jnp.dot(xcat, w1_ref[1], preferred_element_type=jnp.float32)
    acc += jnp.dot(m1[M:2 * M], w1_ref[2],
                   preferred_element_type=jnp.float32)
    r1 = jnp.maximum(acc + b12_ref[0:1], 0.0).astype(jnp.bfloat16)

    m2 = jnp.dot(sb2_ref[...], r1,
                 preferred_element_type=jnp.float32).astype(jnp.bfloat16)
    for g in range(4):
        s = _GS[g]
        a2 = None
        for kh in range(3):
            d = jnp.dot(m2[kh * M:(kh + 1) * M, s:s + 512], w2_ref[kh, g],
                        preferred_element_type=jnp.float32)
            a2 = d if a2 is None else a2 + d
        r2 = jnp.maximum(a2 + b12_ref[1:2, g * 256:(g + 1) * 256], 0.0)
        hm = jnp.maximum(r2[0:M // 2], r2[M // 2:M])
        sh = jnp.concatenate([hm[:, 32:], hm[:, :32]], axis=1)
        o_ref[:, :, g * 256:(g + 1) * 256] = jnp.maximum(hm, sh).astype(
            o_ref.dtype).reshape(16, BB, 256)


def _fc_fused_kernel(x_ref, w3_ref, b3_ref, w4_ref, b4_ref, o_ref, acc_ref):
    k = pl.program_id(1)

    @pl.when(k == 0)
    def _():
        acc_ref[...] = jnp.zeros_like(acc_ref)

    part = jnp.dot(x_ref[0], w3_ref[0], preferred_element_type=jnp.float32)
    for c in range(1, _FC_KC):
        part += jnp.dot(x_ref[c], w3_ref[c],
                        preferred_element_type=jnp.float32)
    acc_ref[...] += part

    @pl.when(k == pl.num_programs(1) - 1)
    def _():
        h = jnp.maximum(acc_ref[...] + b3_ref[...], 0.0).astype(jnp.bfloat16)
        o_ref[...] = (jnp.dot(h, w4_ref[...],
                              preferred_element_type=jnp.float32)
                      + b4_ref[...]).astype(o_ref.dtype)


def _conv_stage(x_nchw, w1L, w2G, b12, sb1, sb2):
    B = x_nchw.shape[0]
    return pl.pallas_call(
        _conv_fused_kernel,
        out_shape=jax.ShapeDtypeStruct((16, B, 1024), jnp.bfloat16),
        grid_spec=pltpu.PrefetchScalarGridSpec(
            num_scalar_prefetch=0,
            grid=(B // _BB,),
            in_specs=[
                pl.BlockSpec((_BB, 3, 32, 32), lambda i: (i, 0, 0, 0)),
                pl.BlockSpec((3, 96, 1024), lambda i: (0, 0, 0)),
                pl.BlockSpec((3, 4, 512, 256), lambda i: (0, 0, 0, 0)),
                pl.BlockSpec((2, 1024), lambda i: (0, 0)),
                pl.BlockSpec((2 * _BB * 32, _BB * 32), lambda i: (0, 0)),
                pl.BlockSpec((3 * _BB * 32, _BB * 32), lambda i: (0, 0)),
            ],
            out_specs=pl.BlockSpec((16, _BB, 1024), lambda i: (0, i, 0)),
        ),
        compiler_params=pltpu.CompilerParams(
            dimension_semantics=("parallel",)),
    )(x_nchw, w1L, w2G, b12, sb1, sb2)


def _fc_stage(feats, w3, b3, w4, b4):
    KC, B, _ = feats.shape
    N3 = w3.shape[2]
    N4 = w4.shape[1]
    return pl.pallas_call(
        _fc_fused_kernel,
        out_shape=jax.ShapeDtypeStruct((B, N4), jnp.float32),
        grid_spec=pltpu.PrefetchScalarGridSpec(
            num_scalar_prefetch=0,
            grid=(B // _FC_BM, KC // _FC_KC),
            in_specs=[
                pl.BlockSpec((_FC_KC, _FC_BM, 1024), lambda i, k: (k, i, 0)),
                pl.BlockSpec((_FC_KC, 1024, N3), lambda i, k: (k, 0, 0)),
                pl.BlockSpec((1, N3), lambda i, k: (0, 0)),
                pl.BlockSpec((N3, N4), lambda i, k: (0, 0)),
                pl.BlockSpec((1, N4), lambda i, k: (0, 0)),
            ],
            out_specs=pl.BlockSpec((_FC_BM, N4), lambda i, k: (i, 0)),
            scratch_shapes=[pltpu.VMEM((_FC_BM, N3), jnp.float32)],
        ),
        compiler_params=pltpu.CompilerParams(
            dimension_semantics=("parallel", "arbitrary")),
    )(feats, w3, b3, w4, b4)


def _shift_consts():
    m = _BB * 32
    sb1 = np.stack([np.kron(np.eye(_BB, dtype=np.float32),
                            np.eye(32, k=kh - 1, dtype=np.float32))
                    for kh in range(3)])
    j = np.arange(m)
    half = j % (m // 2)
    srcrow = (half % _BB) * 32 + (half // _BB) * 2 + j // (m // 2)
    perm = np.zeros((m, m), np.float32)
    perm[j, srcrow] = 1.0
    sb2 = np.einsum('jr,xrs->xjs', perm, sb1)
    sb1_02 = np.concatenate([sb1[0], sb1[2]], axis=0)
    sb2_all = sb2.reshape(3 * m, m)
    return sb1_02, sb2_all


_SB1, _SB2 = _shift_consts()


def _banded(w_taps, cin, row_order):
    wr = w_taps.reshape(3, 3, cin, 32)
    d = jnp.stack([jnp.eye(32, k=1 - kw, dtype=w_taps.dtype)
                   for kw in range(3)])
    wl = jnp.einsum(f'xab,hxio->h{row_order}bo', d, wr)
    return wl.reshape(3, cin * 32, 1024).astype(jnp.bfloat16)


def kernel(conv1_w, conv1_b, conv2_w, conv2_b, fc3_w, fc3_b, fc4_w, fc4_b,
           x_nchw):
    B = x_nchw.shape[0]
    w1L = _banded(conv1_w, 3, 'ia')
    w2L = _banded(conv2_w, 32, 'ai')
    w2G = jnp.stack([jnp.stack([
        jax.lax.dynamic_slice(w2L[kh], (_GS[g], 256 * g), (512, 256))
        for g in range(4)]) for kh in range(3)])
    b12 = jnp.tile(jnp.stack([conv1_b, conv2_b]), (1, 32))
    sb1 = jnp.asarray(_SB1, jnp.bfloat16)
    sb2 = jnp.asarray(_SB2, jnp.bfloat16)

    feats = _conv_stage(x_nchw, w1L, w2G, b12, sb1, sb2)

    wt = fc3_w.reshape(16, 16, 32, 512)
    w3w = jnp.stack([wt, jnp.zeros_like(wt)], axis=2).reshape(16, 1024, 512)

    return _fc_stage(feats, w3w.astype(jnp.bfloat16), fc3_b.reshape(1, 512),
                     fc4_w.astype(jnp.bfloat16), fc4_b.reshape(1, 10))
